# Initial kernel scaffold; baseline (speedup 1.0000x reference)
import jax
import jax.numpy as jnp
from jax.experimental import pallas as pl


def kernel(x, W, a_src, a_dst, edge_index):
    raise NotImplementedError("write your pallas kernel here")


# speedup vs baseline: 375.5093x; 375.5093x over previous
import functools

import jax
import jax.numpy as jnp
from jax import lax
from jax.experimental import pallas as pl
from jax.experimental.pallas import tpu as pltpu
from jax.experimental.pallas import tpu_sc as plsc

IMG_H, IMG_W = 256, 256
N_VERT = IMG_H * IMG_W
T = 4
N_HEADS = 3
ALPHA = 0.2

NC, NS, L = 2, 16, 16
NW = NC * NS
ROWS_PER_W = IMG_H // NW
OWN = ROWS_PER_W * IMG_W
HALO = OWN + 2 * IMG_W
N_PAD = N_VERT + 2 * IMG_W
OUT_D = N_HEADS * T
OWN_OUT = OWN * OUT_D


def _gat_body(xT_hbm, par_hbm, out_hbm, pbuf, xbuf, hbuf, sbuf, dbuf, obuf):
    wid = lax.axis_index("s") * NC + lax.axis_index("c")
    off = wid * OWN

    pltpu.sync_copy(par_hbm, pbuf)
    for t in range(T):
        pltpu.sync_copy(xT_hbm.at[t, pl.ds(off, HALO)], xbuf.at[t])

    iota = lax.iota(jnp.int32, L)

    for head in range(N_HEADS):
        vw = pbuf[pl.ds(head * 32, 16)]
        va = pbuf[pl.ds(head * 32 + 16, 16)]
        w = [[vw[k * 4 + t] for t in range(T)] for k in range(T)]
        a_s = [va[t] for t in range(T)]
        a_d = [va[4 + t] for t in range(T)]

        def ph1(c, _, w=w, a_s=a_s, a_d=a_d, head=head):
            l = c * L
            xv = [xbuf[t, pl.ds(l, L)] for t in range(T)]
            s_acc = None
            d_acc = None
            for t in range(T):
                hv = (xv[0] * w[0][t] + xv[1] * w[1][t]
                      + xv[2] * w[2][t] + xv[3] * w[3][t])
                hbuf[t, pl.ds(l, L)] = hv
                s_acc = hv * a_s[t] if s_acc is None else s_acc + hv * a_s[t]
                d_acc = hv * a_d[t] if d_acc is None else d_acc + hv * a_d[t]
            sbuf[pl.ds(l, L)] = s_acc
            dbuf[pl.ds(l, L)] = d_acc
            return 0

        lax.fori_loop(0, HALO // L, ph1, 0)

        def ph2(c, _, head=head):
            l = IMG_W + c * L
            gv = off + c * L
            jv = jnp.bitwise_and(gv + iota, IMG_W - 1)
            iv = lax.shift_right_logical(gv + iota, 8)
            one = jnp.float32(1.0)
            zero = jnp.float32(0.0)
            m_l = jnp.where(jv == 0, zero, one)
            m_r = jnp.where(jv == IMG_W - 1, zero, one)
            m_u = jnp.where(iv == 0, zero, one)
            m_d = jnp.where(iv == IMG_H - 1, zero, one)

            dv = dbuf[pl.ds(l, L)]
            idx_l = iota + (l - 1)
            idx_r = iota + (l + 1)
            s_by_dir = [
                sbuf[pl.ds(l, L)],
                plsc.load_gather(sbuf, [idx_l]),
                plsc.load_gather(sbuf, [idx_r]),
                sbuf[pl.ds(l - IMG_W, L)],
                sbuf[pl.ds(l + IMG_W, L)],
            ]
            e = []
            for s_src in s_by_dir:
                z = s_src + dv
                e.append(jnp.maximum(z, ALPHA * z))
            m = jnp.maximum(jnp.maximum(jnp.maximum(e[0], e[1]),
                                        jnp.maximum(e[2], e[3])), e[4])
            ex0 = jnp.exp(e[0] - m)
            ex1 = jnp.exp(e[1] - m) * m_l
            ex2 = jnp.exp(e[2] - m) * m_r
            ex3 = jnp.exp(e[3] - m) * m_u
            ex4 = jnp.exp(e[4] - m) * m_d
            inv = one / ((ex0 + ex1) + (ex2 + ex3) + ex4 + jnp.float32(1e-16))
            al = [ex0 * inv, ex1 * inv, ex2 * inv, ex3 * inv, ex4 * inv]

            vidx = (c * L + iota) * OUT_D + (head * T)
            for t in range(T):
                tsplat = jnp.full((L,), t, jnp.int32)
                hv_by_dir = [
                    hbuf[t, pl.ds(l, L)],
                    plsc.load_gather(hbuf, [tsplat, idx_l]),
                    plsc.load_gather(hbuf, [tsplat, idx_r]),
                    hbuf[t, pl.ds(l - IMG_W, L)],
                    hbuf[t, pl.ds(l + IMG_W, L)],
                ]
                o = None
                for k in range(5):
                    o = (al[k] * hv_by_dir[k] if o is None
                         else o + al[k] * hv_by_dir[k])
                o = jnp.where(o > 0, o, jnp.exp(o) - one)
                plsc.store_scatter(obuf, [vidx + t], o)
            return 0

        lax.fori_loop(0, OWN // L, ph2, 0)

    pltpu.sync_copy(obuf, out_hbm.at[pl.ds(wid * OWN_OUT, OWN_OUT)])


@jax.jit
def _gat_sc(xT_pad, params):
    body = functools.partial(
        pl.kernel,
        out_type=jax.ShapeDtypeStruct((N_VERT * OUT_D,), jnp.float32),
        mesh=plsc.VectorSubcoreMesh(
            core_axis_name="c", subcore_axis_name="s",
            num_cores=NC, num_subcores=NS),
        compiler_params=pltpu.CompilerParams(needs_layout_passes=False),
        scratch_types=[
            pltpu.VMEM((96,), jnp.float32),
            pltpu.VMEM((T, HALO), jnp.float32),
            pltpu.VMEM((T, HALO), jnp.float32),
            pltpu.VMEM((HALO,), jnp.float32),
            pltpu.VMEM((HALO,), jnp.float32),
            pltpu.VMEM((OWN_OUT,), jnp.float32),
        ],
    )(_gat_body)
    return body(xT_pad, params)


def kernel(x, W, a_src, a_dst, edge_index):
    xT_pad = jnp.pad(x, ((IMG_W, IMG_W), (0, 0))).T
    pad8 = jnp.zeros((8,), jnp.float32)
    params = jnp.concatenate(
        [jnp.concatenate([W[h].reshape(-1), a_src[h], a_dst[h], pad8])
         for h in range(N_HEADS)])
    out = _gat_sc(xT_pad, params)
    return out.reshape(N_VERT, OUT_D)


# speedup vs baseline: 423.5263x; 1.1279x over previous
import functools

import jax
import jax.numpy as jnp
from jax import lax
from jax.experimental import pallas as pl
from jax.experimental.pallas import tpu as pltpu
from jax.experimental.pallas import tpu_sc as plsc

IMG_H, IMG_W = 256, 256
N_VERT = IMG_H * IMG_W
T = 4
N_HEADS = 3
ALPHA = 0.2

NC, NS, L = 2, 16, 16
NW = NC * NS
ROWS_PER_W = IMG_H // NW
OWN = ROWS_PER_W * IMG_W
HALO = OWN + 2 * IMG_W
N_PAD = N_VERT + 2 * IMG_W
OUT_D = N_HEADS * T
OWN_OUT = OWN * OUT_D


def _gat_body(xT_hbm, par_hbm, out_hbm, pbuf, xbuf,
              h0, h1, h2, s0, s1, s2, d0, d1, d2, obuf):
    hbufs = (h0, h1, h2)
    sbufs = (s0, s1, s2)
    dbufs = (d0, d1, d2)
    wid = lax.axis_index("s") * NC + lax.axis_index("c")
    off = wid * OWN

    pltpu.sync_copy(par_hbm, pbuf)
    for t in range(T):
        pltpu.sync_copy(xT_hbm.at[t, pl.ds(off, HALO)], xbuf.at[t])

    iota = lax.iota(jnp.int32, L)
    one = jnp.float32(1.0)
    zero = jnp.float32(0.0)

    w = []
    a_s = []
    a_d = []
    for head in range(N_HEADS):
        vw = pbuf[pl.ds(head * 32, 16)]
        va = pbuf[pl.ds(head * 32 + 16, 16)]
        w.append([[vw[k * 4 + t] for t in range(T)] for k in range(T)])
        a_s.append([va[t] for t in range(T)])
        a_d.append([va[4 + t] for t in range(T)])

    @plsc.parallel_loop(0, HALO // L, unroll=2)
    def _ph1(c):
        l = c * L
        xv = [xbuf[t, pl.ds(l, L)] for t in range(T)]
        for head in range(N_HEADS):
            s_acc = None
            d_acc = None
            for t in range(T):
                hv = (xv[0] * w[head][0][t] + xv[1] * w[head][1][t]
                      + xv[2] * w[head][2][t] + xv[3] * w[head][3][t])
                hbufs[head][pl.ds(t * HALO + l, L)] = hv
                s_acc = (hv * a_s[head][t] if s_acc is None
                         else s_acc + hv * a_s[head][t])
                d_acc = (hv * a_d[head][t] if d_acc is None
                         else d_acc + hv * a_d[head][t])
            sbufs[head][pl.ds(l, L)] = s_acc
            dbufs[head][pl.ds(l, L)] = d_acc

    @plsc.parallel_loop(0, OWN // L, unroll=2)
    def _ph2(c):
        l = IMG_W + c * L
        gv = off + c * L
        jv = jnp.bitwise_and(gv + iota, IMG_W - 1)
        iv = lax.shift_right_logical(gv + iota, 8)
        m_l = jnp.where(jv == 0, zero, one)
        m_r = jnp.where(jv == IMG_W - 1, zero, one)
        m_u = jnp.where(iv == 0, zero, one)
        m_d = jnp.where(iv == IMG_H - 1, zero, one)
        idx_l = iota + (l - 1)
        idx_r = iota + (l + 1)
        vidx = (c * L + iota) * OUT_D

        for head in range(N_HEADS):
            sbuf = sbufs[head]
            hbuf = hbufs[head]
            dv = dbufs[head][pl.ds(l, L)]
            s_by_dir = [
                sbuf[pl.ds(l, L)],
                plsc.load_gather(sbuf, [idx_l]),
                plsc.load_gather(sbuf, [idx_r]),
                sbuf[pl.ds(l - IMG_W, L)],
                sbuf[pl.ds(l + IMG_W, L)],
            ]
            e = []
            for s_src in s_by_dir:
                z = s_src + dv
                e.append(jnp.maximum(z, ALPHA * z))
            m = jnp.maximum(jnp.maximum(jnp.maximum(e[0], e[1]),
                                        jnp.maximum(e[2], e[3])), e[4])
            ex0 = jnp.exp(e[0] - m)
            ex1 = jnp.exp(e[1] - m) * m_l
            ex2 = jnp.exp(e[2] - m) * m_r
            ex3 = jnp.exp(e[3] - m) * m_u
            ex4 = jnp.exp(e[4] - m) * m_d
            inv = one / ((ex0 + ex1) + (ex2 + ex3) + ex4 + jnp.float32(1e-16))
            al = [ex0 * inv, ex1 * inv, ex2 * inv, ex3 * inv, ex4 * inv]

            for t in range(T):
                hv_by_dir = [
                    hbuf[pl.ds(t * HALO + l, L)],
                    plsc.load_gather(hbuf, [idx_l + (t * HALO)]),
                    plsc.load_gather(hbuf, [idx_r + (t * HALO)]),
                    hbuf[pl.ds(t * HALO + l - IMG_W, L)],
                    hbuf[pl.ds(t * HALO + l + IMG_W, L)],
                ]
                o = None
                for k in range(5):
                    o = (al[k] * hv_by_dir[k] if o is None
                         else o + al[k] * hv_by_dir[k])
                o = jnp.where(o > 0, o, jnp.exp(o) - one)
                plsc.store_scatter(obuf, [vidx + (head * T + t)], o)

    pltpu.sync_copy(obuf, out_hbm.at[pl.ds(wid * OWN_OUT, OWN_OUT)])


@jax.jit
def _gat_sc(xT_pad, params):
    body = functools.partial(
        pl.kernel,
        out_type=jax.ShapeDtypeStruct((N_VERT * OUT_D,), jnp.float32),
        mesh=plsc.VectorSubcoreMesh(
            core_axis_name="c", subcore_axis_name="s",
            num_cores=NC, num_subcores=NS),
        compiler_params=pltpu.CompilerParams(needs_layout_passes=False),
        scratch_types=[
            pltpu.VMEM((96,), jnp.float32),
            pltpu.VMEM((T, HALO), jnp.float32),
            pltpu.VMEM((T * HALO,), jnp.float32),
            pltpu.VMEM((T * HALO,), jnp.float32),
            pltpu.VMEM((T * HALO,), jnp.float32),
            pltpu.VMEM((HALO,), jnp.float32),
            pltpu.VMEM((HALO,), jnp.float32),
            pltpu.VMEM((HALO,), jnp.float32),
            pltpu.VMEM((HALO,), jnp.float32),
            pltpu.VMEM((HALO,), jnp.float32),
            pltpu.VMEM((HALO,), jnp.float32),
            pltpu.VMEM((OWN_OUT,), jnp.float32),
        ],
    )(_gat_body)
    return body(xT_pad, params)


def kernel(x, W, a_src, a_dst, edge_index):
    xT_pad = jnp.pad(x, ((IMG_W, IMG_W), (0, 0))).T
    pad8 = jnp.zeros((8,), jnp.float32)
    params = jnp.concatenate(
        [jnp.concatenate([W[h].reshape(-1), a_src[h], a_dst[h], pad8])
         for h in range(N_HEADS)])
    out = _gat_sc(xT_pad, params)
    return out.reshape(N_VERT, OUT_D)


# speedup vs baseline: 500.4914x; 1.1817x over previous
import functools

import jax
import jax.numpy as jnp
from jax import lax
from jax.experimental import pallas as pl
from jax.experimental.pallas import tpu as pltpu
from jax.experimental.pallas import tpu_sc as plsc

IMG_H, IMG_W = 256, 256
N_VERT = IMG_H * IMG_W
T = 4
N_HEADS = 3
ALPHA = 0.2

NC, NS, L = 2, 16, 16
NW = NC * NS
ROWS_PER_W = IMG_H // NW
OWN = ROWS_PER_W * IMG_W
HALO = OWN + 2 * IMG_W
N_PAD = N_VERT + 2 * IMG_W
OUT_D = N_HEADS * T
OWN_OUT = OWN * OUT_D


def _gat_body(xT_hbm, par_hbm, out_hbm, pbuf, xbuf,
              h0, h1, h2, s0, s1, s2, d0, d1, d2, obuf):
    hbufs = (h0, h1, h2)
    sbufs = (s0, s1, s2)
    dbufs = (d0, d1, d2)
    wid = lax.axis_index("s") * NC + lax.axis_index("c")
    off = wid * OWN

    pltpu.sync_copy(par_hbm, pbuf)
    for t in range(T):
        pltpu.sync_copy(xT_hbm.at[t, pl.ds(off, HALO)], xbuf.at[t])

    iota = lax.iota(jnp.int32, L)
    one = jnp.float32(1.0)
    zero = jnp.float32(0.0)

    w = []
    a_s = []
    a_d = []
    for head in range(N_HEADS):
        vw = pbuf[pl.ds(head * 32, 16)]
        va = pbuf[pl.ds(head * 32 + 16, 16)]
        w.append([[vw[k * 4 + t] for t in range(T)] for k in range(T)])
        a_s.append([va[t] for t in range(T)])
        a_d.append([va[4 + t] for t in range(T)])

    @plsc.parallel_loop(0, HALO // L, unroll=2)
    def _ph1(c):
        l = c * L
        xv = [xbuf[t, pl.ds(l, L)] for t in range(T)]
        for head in range(N_HEADS):
            s_acc = None
            d_acc = None
            for t in range(T):
                hv = (xv[0] * w[head][0][t] + xv[1] * w[head][1][t]
                      + xv[2] * w[head][2][t] + xv[3] * w[head][3][t])
                hbufs[head][pl.ds(t * HALO + l, L)] = hv
                s_acc = (hv * a_s[head][t] if s_acc is None
                         else s_acc + hv * a_s[head][t])
                d_acc = (hv * a_d[head][t] if d_acc is None
                         else d_acc + hv * a_d[head][t])
            sbufs[head][pl.ds(l, L)] = s_acc
            dbufs[head][pl.ds(l, L)] = d_acc

    @plsc.parallel_loop(0, OWN // L, unroll=2)
    def _ph2(c):
        l = IMG_W + c * L
        gv = off + c * L
        jv = jnp.bitwise_and(gv + iota, IMG_W - 1)
        iv = lax.shift_right_logical(gv + iota, 8)
        m_l = jnp.where(jv == 0, zero, one)
        m_r = jnp.where(jv == IMG_W - 1, zero, one)
        m_u = jnp.where(iv == 0, zero, one)
        m_d = jnp.where(iv == IMG_H - 1, zero, one)
        idx_l = iota + (l - 1)
        idx_r = iota + (l + 1)
        vidx = (c * L + iota) * 16

        for head in range(N_HEADS):
            sbuf = sbufs[head]
            hbuf = hbufs[head]
            dv = dbufs[head][pl.ds(l, L)]
            s_by_dir = [
                sbuf[pl.ds(l, L)],
                plsc.load_gather(sbuf, [idx_l]),
                plsc.load_gather(sbuf, [idx_r]),
                sbuf[pl.ds(l - IMG_W, L)],
                sbuf[pl.ds(l + IMG_W, L)],
            ]
            e = []
            for s_src in s_by_dir:
                z = s_src + dv
                e.append(jnp.maximum(z, ALPHA * z))
            m = jnp.maximum(jnp.maximum(jnp.maximum(e[0], e[1]),
                                        jnp.maximum(e[2], e[3])), e[4])
            ex0 = jnp.exp(e[0] - m)
            ex1 = jnp.exp(e[1] - m) * m_l
            ex2 = jnp.exp(e[2] - m) * m_r
            ex3 = jnp.exp(e[3] - m) * m_u
            ex4 = jnp.exp(e[4] - m) * m_d
            inv = one / ((ex0 + ex1) + (ex2 + ex3) + ex4 + jnp.float32(1e-16))
            al = [ex0 * inv, ex1 * inv, ex2 * inv, ex3 * inv, ex4 * inv]

            for t in range(T):
                hv_by_dir = [
                    hbuf[pl.ds(t * HALO + l, L)],
                    plsc.load_gather(hbuf, [idx_l + (t * HALO)]),
                    plsc.load_gather(hbuf, [idx_r + (t * HALO)]),
                    hbuf[pl.ds(t * HALO + l - IMG_W, L)],
                    hbuf[pl.ds(t * HALO + l + IMG_W, L)],
                ]
                o = None
                for k in range(5):
                    o = (al[k] * hv_by_dir[k] if o is None
                         else o + al[k] * hv_by_dir[k])
                o = jnp.where(o > 0, o, jnp.exp(o) - one)
                plsc.store_scatter(obuf, [vidx + (head * T + t)], o)

    pltpu.sync_copy(obuf, out_hbm.at[pl.ds(wid * OWN * 16, OWN * 16)])


@jax.jit
def _gat_sc(xT_pad, params):
    body = functools.partial(
        pl.kernel,
        out_type=jax.ShapeDtypeStruct((N_VERT * 16,), jnp.float32),
        mesh=plsc.VectorSubcoreMesh(
            core_axis_name="c", subcore_axis_name="s",
            num_cores=NC, num_subcores=NS),
        compiler_params=pltpu.CompilerParams(needs_layout_passes=False),
        scratch_types=[
            pltpu.VMEM((96,), jnp.float32),
            pltpu.VMEM((T, HALO), jnp.float32),
            pltpu.VMEM((T * HALO,), jnp.float32),
            pltpu.VMEM((T * HALO,), jnp.float32),
            pltpu.VMEM((T * HALO,), jnp.float32),
            pltpu.VMEM((HALO,), jnp.float32),
            pltpu.VMEM((HALO,), jnp.float32),
            pltpu.VMEM((HALO,), jnp.float32),
            pltpu.VMEM((HALO,), jnp.float32),
            pltpu.VMEM((HALO,), jnp.float32),
            pltpu.VMEM((HALO,), jnp.float32),
            pltpu.VMEM((OWN * 16,), jnp.float32),
        ],
    )(_gat_body)
    return body(xT_pad, params)


def kernel(x, W, a_src, a_dst, edge_index):
    xT_pad = jnp.pad(x, ((IMG_W, IMG_W), (0, 0))).T
    pad8 = jnp.zeros((8,), jnp.float32)
    params = jnp.concatenate(
        [jnp.concatenate([W[h].reshape(-1), a_src[h], a_dst[h], pad8])
         for h in range(N_HEADS)])
    return _gat_sc(xT_pad, params).reshape(N_VERT, 16)[:, :OUT_D]


# speedup vs baseline: 781.9538x; 1.5624x over previous
import functools

import jax
import jax.numpy as jnp
from jax import lax
from jax.experimental import pallas as pl
from jax.experimental.pallas import tpu as pltpu
from jax.experimental.pallas import tpu_sc as plsc

IMG_H, IMG_W = 256, 256
N_VERT = IMG_H * IMG_W
T = 4
N_HEADS = 3
ALPHA = 0.2

NC, NS, L = 2, 16, 16
NW = NC * NS
ROWS_PER_W = IMG_H // NW
OWN = ROWS_PER_W * IMG_W
HALO = OWN + 2 * IMG_W
N_PAD = N_VERT + 2 * IMG_W
OUT_D = N_HEADS * T
OWN_OUT = OWN * OUT_D


def _gat_body(xT_hbm, par_hbm, out_hbm, pbuf, xbuf,
              h0, h1, h2, s0, s1, s2, d0, d1, d2, obuf):
    hbufs = (h0, h1, h2)
    sbufs = (s0, s1, s2)
    dbufs = (d0, d1, d2)
    wid = lax.axis_index("s") * NC + lax.axis_index("c")
    off = wid * OWN

    pltpu.sync_copy(par_hbm, pbuf)
    for t in range(T):
        pltpu.sync_copy(xT_hbm.at[t, pl.ds(off, HALO)], xbuf.at[t])

    iota = lax.iota(jnp.int32, L)
    one = jnp.float32(1.0)
    zero = jnp.float32(0.0)

    w = []
    a_s = []
    a_d = []
    for head in range(N_HEADS):
        vw = pbuf[pl.ds(head * 32, 16)]
        va = pbuf[pl.ds(head * 32 + 16, 16)]
        w.append([[vw[k * 4 + t] for t in range(T)] for k in range(T)])
        a_s.append([va[t] for t in range(T)])
        a_d.append([va[4 + t] for t in range(T)])

    @plsc.parallel_loop(0, HALO // L, unroll=2)
    def _ph1(c):
        l = c * L
        xv = [xbuf[t, pl.ds(l, L)] for t in range(T)]
        for head in range(N_HEADS):
            s_acc = None
            d_acc = None
            for t in range(T):
                hv = (xv[0] * w[head][0][t] + xv[1] * w[head][1][t]
                      + xv[2] * w[head][2][t] + xv[3] * w[head][3][t])
                hbufs[head][pl.ds(t * HALO + l, L)] = hv
                s_acc = (hv * a_s[head][t] if s_acc is None
                         else s_acc + hv * a_s[head][t])
                d_acc = (hv * a_d[head][t] if d_acc is None
                         else d_acc + hv * a_d[head][t])
            sbufs[head][pl.ds(l, L)] = s_acc
            dbufs[head][pl.ds(l, L)] = d_acc

    @plsc.parallel_loop(0, OWN // L, unroll=2)
    def _ph2(c):
        l = IMG_W + c * L
        gv = off + c * L
        jv = jnp.bitwise_and(gv + iota, IMG_W - 1)
        iv = lax.shift_right_logical(gv + iota, 8)
        m_l = jnp.where(jv == 0, zero, one)
        m_r = jnp.where(jv == IMG_W - 1, zero, one)
        m_u = jnp.where(iv == 0, zero, one)
        m_d = jnp.where(iv == IMG_H - 1, zero, one)
        idx_l = iota + (l - 1)
        idx_r = iota + (l + 1)
        row_idx = c * L + iota

        for head in range(N_HEADS):
            sbuf = sbufs[head]
            hbuf = hbufs[head]
            dv = dbufs[head][pl.ds(l, L)]
            s_by_dir = [
                sbuf[pl.ds(l, L)],
                plsc.load_gather(sbuf, [idx_l]),
                plsc.load_gather(sbuf, [idx_r]),
                sbuf[pl.ds(l - IMG_W, L)],
                sbuf[pl.ds(l + IMG_W, L)],
            ]
            e = []
            for s_src in s_by_dir:
                z = s_src + dv
                e.append(jnp.maximum(z, ALPHA * z))
            m = jnp.maximum(jnp.maximum(jnp.maximum(e[0], e[1]),
                                        jnp.maximum(e[2], e[3])), e[4])
            ex0 = jnp.exp(e[0] - m)
            ex1 = jnp.exp(e[1] - m) * m_l
            ex2 = jnp.exp(e[2] - m) * m_r
            ex3 = jnp.exp(e[3] - m) * m_u
            ex4 = jnp.exp(e[4] - m) * m_d
            inv = one / ((ex0 + ex1) + (ex2 + ex3) + ex4 + jnp.float32(1e-16))
            al = [ex0 * inv, ex1 * inv, ex2 * inv, ex3 * inv, ex4 * inv]

            for t in range(T):
                hv_by_dir = [
                    hbuf[pl.ds(t * HALO + l, L)],
                    plsc.load_gather(hbuf, [idx_l + (t * HALO)]),
                    plsc.load_gather(hbuf, [idx_r + (t * HALO)]),
                    hbuf[pl.ds(t * HALO + l - IMG_W, L)],
                    hbuf[pl.ds(t * HALO + l + IMG_W, L)],
                ]
                o = None
                for k in range(5):
                    o = (al[k] * hv_by_dir[k] if o is None
                         else o + al[k] * hv_by_dir[k])
                o = jnp.where(o > 0, o, jnp.exp(o) - one)
                plsc.store_scatter(obuf, [row_idx + (head * T + t) * OWN], o)

    for cc in range(OUT_D):
        pltpu.sync_copy(obuf.at[pl.ds(cc * OWN, OWN)],
                        out_hbm.at[pl.ds(cc * N_VERT + wid * OWN, OWN)])


@jax.jit
def _gat_sc(xT_pad, params):
    body = functools.partial(
        pl.kernel,
        out_type=jax.ShapeDtypeStruct((OUT_D * N_VERT,), jnp.float32),
        mesh=plsc.VectorSubcoreMesh(
            core_axis_name="c", subcore_axis_name="s",
            num_cores=NC, num_subcores=NS),
        compiler_params=pltpu.CompilerParams(needs_layout_passes=False),
        scratch_types=[
            pltpu.VMEM((96,), jnp.float32),
            pltpu.VMEM((T, HALO), jnp.float32),
            pltpu.VMEM((T * HALO,), jnp.float32),
            pltpu.VMEM((T * HALO,), jnp.float32),
            pltpu.VMEM((T * HALO,), jnp.float32),
            pltpu.VMEM((HALO,), jnp.float32),
            pltpu.VMEM((HALO,), jnp.float32),
            pltpu.VMEM((HALO,), jnp.float32),
            pltpu.VMEM((HALO,), jnp.float32),
            pltpu.VMEM((HALO,), jnp.float32),
            pltpu.VMEM((HALO,), jnp.float32),
            pltpu.VMEM((OUT_D * OWN,), jnp.float32),
        ],
    )(_gat_body)
    return body(xT_pad, params)


def kernel(x, W, a_src, a_dst, edge_index):
    xT_pad = jnp.pad(x, ((IMG_W, IMG_W), (0, 0))).T
    pad8 = jnp.zeros((8,), jnp.float32)
    params = jnp.concatenate(
        [jnp.concatenate([W[h].reshape(-1), a_src[h], a_dst[h], pad8])
         for h in range(N_HEADS)])
    return _gat_sc(xT_pad, params).reshape(OUT_D, N_VERT).T
